# Initial kernel scaffold; baseline (speedup 1.0000x reference)
import jax
import jax.numpy as jnp
from jax.experimental import pallas as pl


def kernel(positions, edge_index):
    raise NotImplementedError("write your pallas kernel here")


# speedup vs baseline: 2.7055x; 2.7055x over previous
import functools
import math

import jax
import jax.numpy as jnp
from jax import lax
from jax.experimental import pallas as pl
from jax.experimental.pallas import tpu as pltpu
from jax.experimental.pallas import tpu_sc as plsc

R_CUT = 5.0
N_MAX = 4
GAMMA = (N_MAX / R_CUT) ** 2
R_N = tuple(i * R_CUT / (N_MAX - 1) for i in range(N_MAX))
C0 = 0.28209479177387814
C1 = 0.4886025119029199
C2A = 1.0925484305920792
C2C = 0.31539156525252005
C2E = 0.5462742152960396
HALF_PI = math.pi / 2.0
U_SCALE = math.pi / (2.0 * R_CUT)
COS_COEFFS = (-1.0 / 87178291200.0, 1.0 / 479001600.0, -1.0 / 3628800.0,
              1.0 / 40320.0, -1.0 / 720.0, 1.0 / 24.0, -0.5, 1.0)

CHUNK = 128
LANES = 16
N_OUT = 36


def _expand_16(cbuf, nbuf, v):
    sl = pl.ds(v * LANES, LANES)
    x = nbuf[0][sl] - cbuf[0][sl]
    y = nbuf[1][sl] - cbuf[1][sl]
    z = nbuf[2][sl] - cbuf[2][sl]
    r2 = x * x + y * y + z * z + jnp.float32(1e-12)
    seed = (jnp.int32(0x5F3759DF)
            - (lax.bitcast_convert_type(r2, jnp.int32) >> 1))
    rinv = lax.bitcast_convert_type(seed, jnp.float32)
    for _ in range(3):
        rinv = rinv * (jnp.float32(1.5) - jnp.float32(0.5) * r2 * rinv * rinv)
    r = r2 * rinv
    rinv2 = rinv * rinv
    h = jnp.minimum(jnp.float32(U_SCALE) * r, jnp.float32(HALF_PI))
    h2 = h * h
    cc = jnp.full((LANES,), COS_COEFFS[0], jnp.float32)
    for co in COS_COEFFS[1:]:
        cc = cc * h2 + jnp.float32(co)
    fcut = jnp.where(r < jnp.float32(R_CUT), cc * cc, jnp.float32(0.0))
    rad = []
    for rn in R_N:
        dr = r - jnp.float32(rn)
        rad.append(jnp.exp(-jnp.float32(GAMMA) * dr * dr) * fcut)
    sh = [jnp.full((LANES,), C0, jnp.float32),
          jnp.float32(C1) * y * rinv,
          jnp.float32(C1) * z * rinv,
          jnp.float32(C1) * x * rinv,
          jnp.float32(C2A) * x * y * rinv2,
          jnp.float32(C2A) * y * z * rinv2,
          jnp.float32(C2C) * (jnp.float32(2.0) * z * z - x * x - y * y) * rinv2,
          jnp.float32(C2A) * x * z * rinv2,
          jnp.float32(C2E) * (x * x - y * y) * rinv2]
    return sh, rad


def _sc_body(pos_x, pos_y, pos_z, cen_hbm, nbr_hbm, out_hbm,
             idx_c, idx_n, cx, cy, cz, nx, ny, nz, out_stage, sem,
             *, n_rows, n_workers):
    nc = plsc.get_sparse_core_info().num_cores
    w = lax.axis_index("s") * nc + lax.axis_index("c")
    nk = (jnp.int32(n_rows - 1) - w) // jnp.int32(n_workers) + 1
    cbuf = (cx, cy, cz)
    nbuf = (nx, ny, nz)

    @pl.loop(0, nk)
    def _row(k):
        row = w + k * jnp.int32(n_workers)
        pltpu.sync_copy(cen_hbm.at[row], idx_c)
        pltpu.sync_copy(nbr_hbm.at[row], idx_n)
        gathers = []
        for tab, dst in zip((pos_x, pos_y, pos_z), cbuf):
            gathers.append(pltpu.async_copy(tab.at[idx_c], dst, sem))
        for tab, dst in zip((pos_x, pos_y, pos_z), nbuf):
            gathers.append(pltpu.async_copy(tab.at[idx_n], dst, sem))
        for g in gathers:
            g.wait()
        for v in range(CHUNK // LANES):
            sh, rad = _expand_16(cbuf, nbuf, v)
            oidx = (lax.iota(jnp.int32, LANES) + jnp.int32(v * LANES)) \
                * jnp.int32(N_OUT)
            one = jnp.full((LANES,), 1, jnp.int32)
            for s in sh:
                for rd in rad:
                    plsc.store_scatter(out_stage, [oidx], s * rd)
                    oidx = oidx + one
        pltpu.sync_copy(out_stage, out_hbm.at[row])


def kernel(positions, edge_index):
    n_edges = edge_index.shape[1]
    assert n_edges % CHUNK == 0
    n_rows = n_edges // CHUNK

    pos_x = positions[:, 0]
    pos_y = positions[:, 1]
    pos_z = positions[:, 2]
    cen = edge_index[0].reshape(n_rows, CHUNK)
    nbr = edge_index[1].reshape(n_rows, CHUNK)

    info = plsc.get_sparse_core_info()
    n_workers = info.num_cores * info.num_subcores
    mesh = plsc.VectorSubcoreMesh(core_axis_name="c", subcore_axis_name="s")
    body = functools.partial(_sc_body, n_rows=n_rows, n_workers=n_workers)
    out = pl.kernel(
        body,
        out_type=jax.ShapeDtypeStruct((n_rows, CHUNK * N_OUT), jnp.float32),
        mesh=mesh,
        scratch_types=[
            pltpu.VMEM((CHUNK,), jnp.int32),
            pltpu.VMEM((CHUNK,), jnp.int32),
            pltpu.VMEM((CHUNK,), jnp.float32),
            pltpu.VMEM((CHUNK,), jnp.float32),
            pltpu.VMEM((CHUNK,), jnp.float32),
            pltpu.VMEM((CHUNK,), jnp.float32),
            pltpu.VMEM((CHUNK,), jnp.float32),
            pltpu.VMEM((CHUNK,), jnp.float32),
            pltpu.VMEM((CHUNK * N_OUT,), jnp.float32),
            pltpu.SemaphoreType.DMA,
        ],
        compiler_params=pltpu.CompilerParams(needs_layout_passes=False),
    )(pos_x, pos_y, pos_z, cen, nbr)
    return out.reshape(n_edges, N_OUT)


# speedup vs baseline: 3.3461x; 1.2368x over previous
import functools
import math

import jax
import jax.numpy as jnp
from jax import lax
from jax.experimental import pallas as pl
from jax.experimental.pallas import tpu as pltpu
from jax.experimental.pallas import tpu_sc as plsc

R_CUT = 5.0
N_MAX = 4
GAMMA = (N_MAX / R_CUT) ** 2
R_N = tuple(i * R_CUT / (N_MAX - 1) for i in range(N_MAX))
C0 = 0.28209479177387814
C1 = 0.4886025119029199
C2A = 1.0925484305920792
C2C = 0.31539156525252005
C2E = 0.5462742152960396
HALF_PI = math.pi / 2.0
U_SCALE = math.pi / (2.0 * R_CUT)
COS_COEFFS = (-1.0 / 87178291200.0, 1.0 / 479001600.0, -1.0 / 3628800.0,
              1.0 / 40320.0, -1.0 / 720.0, 1.0 / 24.0, -0.5, 1.0)

CHUNK = 128
LANES = 16
N_OUT = 36


def _expand_16(cbuf, nbuf, v):
    sl = pl.ds(v * LANES, LANES)
    x = nbuf[0][sl] - cbuf[0][sl]
    y = nbuf[1][sl] - cbuf[1][sl]
    z = nbuf[2][sl] - cbuf[2][sl]
    r2 = x * x + y * y + z * z + jnp.float32(1e-12)
    seed = (jnp.int32(0x5F3759DF)
            - (lax.bitcast_convert_type(r2, jnp.int32) >> 1))
    rinv = lax.bitcast_convert_type(seed, jnp.float32)
    for _ in range(3):
        rinv = rinv * (jnp.float32(1.5) - jnp.float32(0.5) * r2 * rinv * rinv)
    r = r2 * rinv
    rinv2 = rinv * rinv
    h = jnp.minimum(jnp.float32(U_SCALE) * r, jnp.float32(HALF_PI))
    h2 = h * h
    cc = jnp.full((LANES,), COS_COEFFS[0], jnp.float32)
    for co in COS_COEFFS[1:]:
        cc = cc * h2 + jnp.float32(co)
    fcut = jnp.where(r < jnp.float32(R_CUT), cc * cc, jnp.float32(0.0))
    rad = []
    for rn in R_N:
        dr = r - jnp.float32(rn)
        rad.append(jnp.exp(-jnp.float32(GAMMA) * dr * dr) * fcut)
    sh = [jnp.full((LANES,), C0, jnp.float32),
          jnp.float32(C1) * y * rinv,
          jnp.float32(C1) * z * rinv,
          jnp.float32(C1) * x * rinv,
          jnp.float32(C2A) * x * y * rinv2,
          jnp.float32(C2A) * y * z * rinv2,
          jnp.float32(C2C) * (jnp.float32(2.0) * z * z - x * x - y * y) * rinv2,
          jnp.float32(C2A) * x * z * rinv2,
          jnp.float32(C2E) * (x * x - y * y) * rinv2]
    return sh, rad


def _compute_chunk(cbuf, nbuf, out_stage):
    for v in range(CHUNK // LANES):
        sh, rad = _expand_16(cbuf, nbuf, v)
        oidx = (lax.iota(jnp.int32, LANES) + jnp.int32(v * LANES)) \
            * jnp.int32(N_OUT)
        one = jnp.full((LANES,), 1, jnp.int32)
        for s in sh:
            for rd in rad:
                plsc.store_scatter(out_stage, [oidx], s * rd)
                oidx = oidx + one


def _sc_body(pos_x, pos_y, pos_z, cen_hbm, nbr_hbm, out_hbm,
             ic0, ic1, in0, in1,
             c0x, c0y, c0z, n0x, n0y, n0z,
             c1x, c1y, c1z, n1x, n1y, n1z,
             os0, os1, si0, si1, sg0, sg1, so0, so1,
             *, n_rows, n_workers):
    nc = plsc.get_sparse_core_info().num_cores
    w = lax.axis_index("s") * nc + lax.axis_index("c")
    nk = (jnp.int32(n_rows - 1) - w) // jnp.int32(n_workers) + 1

    idx_c = (ic0, ic1)
    idx_n = (in0, in1)
    cbuf = ((c0x, c0y, c0z), (c1x, c1y, c1z))
    nbuf = ((n0x, n0y, n0z), (n1x, n1y, n1z))
    ostg = (os0, os1)
    sem_i = (si0, si1)
    sem_g = (sg0, sg1)
    sem_o = (so0, so1)
    tabs = (pos_x, pos_y, pos_z)

    def row_of(k):
        return w + k * jnp.int32(n_workers)

    def issue_idx(k, s):
        pltpu.async_copy(cen_hbm.at[row_of(k)], idx_c[s], sem_i[s])
        pltpu.async_copy(nbr_hbm.at[row_of(k)], idx_n[s], sem_i[s])

    def wait_idx(s):
        pltpu.make_async_copy(cen_hbm.at[0], idx_c[s], sem_i[s]).wait()
        pltpu.make_async_copy(nbr_hbm.at[0], idx_n[s], sem_i[s]).wait()

    def issue_gathers(s):
        for tab, dst in zip(tabs, cbuf[s]):
            pltpu.async_copy(tab.at[idx_c[s]], dst, sem_g[s])
        for tab, dst in zip(tabs, nbuf[s]):
            pltpu.async_copy(tab.at[idx_n[s]], dst, sem_g[s])

    def wait_gathers(s):
        for tab, dst in zip(tabs, cbuf[s]):
            pltpu.make_async_copy(tab.at[idx_c[s]], dst, sem_g[s]).wait()
        for tab, dst in zip(tabs, nbuf[s]):
            pltpu.make_async_copy(tab.at[idx_n[s]], dst, sem_g[s]).wait()

    def issue_out(k, s):
        pltpu.async_copy(ostg[s], out_hbm.at[row_of(k)], sem_o[s])

    def wait_out(s):
        pltpu.make_async_copy(ostg[s], out_hbm.at[0], sem_o[s]).wait()

    issue_idx(0, 0)

    @pl.when(nk > 1)
    def _():
        issue_idx(1, 1)

    wait_idx(0)
    issue_gathers(0)

    @pl.loop(0, nk, step=2)
    def _pair(k0):
        for s in (0, 1):
            k = k0 + jnp.int32(s)

            @pl.when(k < nk)
            def _():
                wait_gathers(s)

                @pl.when(k + 2 < nk)
                def _():
                    issue_idx(k + 2, s)

                @pl.when(k + 1 < nk)
                def _():
                    wait_idx(1 - s)
                    issue_gathers(1 - s)

                @pl.when(k >= 2)
                def _():
                    wait_out(s)

                _compute_chunk(cbuf[s], nbuf[s], ostg[s])
                issue_out(k, s)

    @pl.when(nk >= 2)
    def _():
        wait_out(0)
        wait_out(1)

    @pl.when(nk == 1)
    def _():
        wait_out(0)


def kernel(positions, edge_index):
    n_edges = edge_index.shape[1]
    assert n_edges % CHUNK == 0
    n_rows = n_edges // CHUNK

    pos_x = positions[:, 0]
    pos_y = positions[:, 1]
    pos_z = positions[:, 2]
    cen = edge_index[0].reshape(n_rows, CHUNK)
    nbr = edge_index[1].reshape(n_rows, CHUNK)

    info = plsc.get_sparse_core_info()
    n_workers = info.num_cores * info.num_subcores
    mesh = plsc.VectorSubcoreMesh(core_axis_name="c", subcore_axis_name="s")
    body = functools.partial(_sc_body, n_rows=n_rows, n_workers=n_workers)
    idx_t = pltpu.VMEM((CHUNK,), jnp.int32)
    buf_t = pltpu.VMEM((CHUNK,), jnp.float32)
    out_t = pltpu.VMEM((CHUNK * N_OUT,), jnp.float32)
    out = pl.kernel(
        body,
        out_type=jax.ShapeDtypeStruct((n_rows, CHUNK * N_OUT), jnp.float32),
        mesh=mesh,
        scratch_types=(
            [idx_t] * 4 + [buf_t] * 12 + [out_t] * 2
            + [pltpu.SemaphoreType.DMA] * 6
        ),
        compiler_params=pltpu.CompilerParams(needs_layout_passes=False),
    )(pos_x, pos_y, pos_z, cen, nbr)
    return out.reshape(n_edges, N_OUT)


# speedup vs baseline: 3.5271x; 1.0541x over previous
import functools
import math

import jax
import jax.numpy as jnp
from jax import lax
from jax.experimental import pallas as pl
from jax.experimental.pallas import tpu as pltpu
from jax.experimental.pallas import tpu_sc as plsc

R_CUT = 5.0
N_MAX = 4
GAMMA = (N_MAX / R_CUT) ** 2
R_N = tuple(i * R_CUT / (N_MAX - 1) for i in range(N_MAX))
C0 = 0.28209479177387814
C1 = 0.4886025119029199
C2A = 1.0925484305920792
C2C = 0.31539156525252005
C2E = 0.5462742152960396
HALF_PI = math.pi / 2.0
U_SCALE = math.pi / (2.0 * R_CUT)
COS_COEFFS = (-1.0 / 87178291200.0, 1.0 / 479001600.0, -1.0 / 3628800.0,
              1.0 / 40320.0, -1.0 / 720.0, 1.0 / 24.0, -0.5, 1.0)

CHUNK = 128
LANES = 16
N_OUT = 36


def _expand_16(cbuf, nbuf, v):
    sl = pl.ds(v * LANES, LANES)
    x = nbuf[0][sl] - cbuf[0][sl]
    y = nbuf[1][sl] - cbuf[1][sl]
    z = nbuf[2][sl] - cbuf[2][sl]
    r2 = x * x + y * y + z * z + jnp.float32(1e-12)
    seed = (jnp.int32(0x5F3759DF)
            - (lax.bitcast_convert_type(r2, jnp.int32) >> 1))
    rinv = lax.bitcast_convert_type(seed, jnp.float32)
    for _ in range(3):
        rinv = rinv * (jnp.float32(1.5) - jnp.float32(0.5) * r2 * rinv * rinv)
    r = r2 * rinv
    rinv2 = rinv * rinv
    h = jnp.minimum(jnp.float32(U_SCALE) * r, jnp.float32(HALF_PI))
    h2 = h * h
    cc = jnp.full((LANES,), COS_COEFFS[0], jnp.float32)
    for co in COS_COEFFS[1:]:
        cc = cc * h2 + jnp.float32(co)
    fcut = jnp.where(r < jnp.float32(R_CUT), cc * cc, jnp.float32(0.0))
    rad = []
    for rn in R_N:
        dr = r - jnp.float32(rn)
        rad.append(jnp.exp(-jnp.float32(GAMMA) * dr * dr) * fcut)
    sh = [jnp.full((LANES,), C0, jnp.float32),
          jnp.float32(C1) * y * rinv,
          jnp.float32(C1) * z * rinv,
          jnp.float32(C1) * x * rinv,
          jnp.float32(C2A) * x * y * rinv2,
          jnp.float32(C2A) * y * z * rinv2,
          jnp.float32(C2C) * (jnp.float32(2.0) * z * z - x * x - y * y) * rinv2,
          jnp.float32(C2A) * x * z * rinv2,
          jnp.float32(C2E) * (x * x - y * y) * rinv2]
    return sh, rad


def _compute_chunk(cbuf, nbuf, out_stage):
    for v in range(CHUNK // LANES):
        sh, rad = _expand_16(cbuf, nbuf, v)
        oidx = (lax.iota(jnp.int32, LANES) + jnp.int32(v * LANES)) \
            * jnp.int32(N_OUT)
        one = jnp.full((LANES,), 1, jnp.int32)
        for s in sh:
            for rd in rad:
                plsc.store_scatter(out_stage, [oidx], s * rd)
                oidx = oidx + one


def _sc_body(pos_x, pos_y, pos_z, cen_hbm, nbr_hbm, out_hbm,
             ic0, ic1, in0, in1,
             c0x, c0y, c0z, n0x, n0y, n0z,
             c1x, c1y, c1z, n1x, n1y, n1z,
             os0, os1, shx, shy, shz,
             si0, si1, sg0, sg1, so0, so1, st0,
             *, n_rows, n_workers):
    nc = plsc.get_sparse_core_info().num_cores
    w = lax.axis_index("s") * nc + lax.axis_index("c")
    nk = (jnp.int32(n_rows - 1) - w) // jnp.int32(n_workers) + 1

    idx_c = (ic0, ic1)
    idx_n = (in0, in1)
    cbuf = ((c0x, c0y, c0z), (c1x, c1y, c1z))
    nbuf = ((n0x, n0y, n0z), (n1x, n1y, n1z))
    ostg = (os0, os1)
    sem_i = (si0, si1)
    sem_g = (sg0, sg1)
    sem_o = (so0, so1)
    tabs = (shx, shy, shz)

    @pl.when(lax.axis_index("s") == 0)
    def _():
        pltpu.async_copy(pos_x, shx, st0)
        pltpu.async_copy(pos_y, shy, st0)
        pltpu.async_copy(pos_z, shz, st0)
        pltpu.make_async_copy(pos_x, shx, st0).wait()
        pltpu.make_async_copy(pos_y, shy, st0).wait()
        pltpu.make_async_copy(pos_z, shz, st0).wait()

    plsc.subcore_barrier()

    def row_of(k):
        return w + k * jnp.int32(n_workers)

    def issue_idx(k, s):
        pltpu.async_copy(cen_hbm.at[row_of(k)], idx_c[s], sem_i[s])
        pltpu.async_copy(nbr_hbm.at[row_of(k)], idx_n[s], sem_i[s])

    def wait_idx(s):
        pltpu.make_async_copy(cen_hbm.at[0], idx_c[s], sem_i[s]).wait()
        pltpu.make_async_copy(nbr_hbm.at[0], idx_n[s], sem_i[s]).wait()

    def issue_gathers(s):
        for tab, dst in zip(tabs, cbuf[s]):
            pltpu.async_copy(tab.at[idx_c[s]], dst, sem_g[s])
        for tab, dst in zip(tabs, nbuf[s]):
            pltpu.async_copy(tab.at[idx_n[s]], dst, sem_g[s])

    def wait_gathers(s):
        for tab, dst in zip(tabs, cbuf[s]):
            pltpu.make_async_copy(tab.at[idx_c[s]], dst, sem_g[s]).wait()
        for tab, dst in zip(tabs, nbuf[s]):
            pltpu.make_async_copy(tab.at[idx_n[s]], dst, sem_g[s]).wait()

    def issue_out(k, s):
        pltpu.async_copy(ostg[s], out_hbm.at[row_of(k)], sem_o[s])

    def wait_out(s):
        pltpu.make_async_copy(ostg[s], out_hbm.at[0], sem_o[s]).wait()

    issue_idx(0, 0)

    @pl.when(nk > 1)
    def _():
        issue_idx(1, 1)

    wait_idx(0)
    issue_gathers(0)

    @pl.loop(0, nk, step=2)
    def _pair(k0):
        for s in (0, 1):
            k = k0 + jnp.int32(s)

            @pl.when(k < nk)
            def _():
                wait_gathers(s)

                @pl.when(k + 2 < nk)
                def _():
                    issue_idx(k + 2, s)

                @pl.when(k + 1 < nk)
                def _():
                    wait_idx(1 - s)
                    issue_gathers(1 - s)

                @pl.when(k >= 2)
                def _():
                    wait_out(s)

                _compute_chunk(cbuf[s], nbuf[s], ostg[s])
                issue_out(k, s)

    @pl.when(nk >= 2)
    def _():
        wait_out(0)
        wait_out(1)

    @pl.when(nk == 1)
    def _():
        wait_out(0)


def kernel(positions, edge_index):
    n_edges = edge_index.shape[1]
    assert n_edges % CHUNK == 0
    n_rows = n_edges // CHUNK

    pos_x = positions[:, 0]
    pos_y = positions[:, 1]
    pos_z = positions[:, 2]
    cen = edge_index[0].reshape(n_rows, CHUNK)
    nbr = edge_index[1].reshape(n_rows, CHUNK)

    info = plsc.get_sparse_core_info()
    n_workers = info.num_cores * info.num_subcores
    mesh = plsc.VectorSubcoreMesh(core_axis_name="c", subcore_axis_name="s")
    body = functools.partial(_sc_body, n_rows=n_rows, n_workers=n_workers)
    idx_t = pltpu.VMEM((CHUNK,), jnp.int32)
    buf_t = pltpu.VMEM((CHUNK,), jnp.float32)
    out_t = pltpu.VMEM((CHUNK * N_OUT,), jnp.float32)
    out = pl.kernel(
        body,
        out_type=jax.ShapeDtypeStruct((n_rows, CHUNK * N_OUT), jnp.float32),
        mesh=mesh,
        scratch_types=(
            [idx_t] * 4 + [buf_t] * 12 + [out_t] * 2
            + [pltpu.VMEM_SHARED((positions.shape[0],), jnp.float32)] * 3
            + [pltpu.SemaphoreType.DMA] * 7
        ),
        compiler_params=pltpu.CompilerParams(needs_layout_passes=False),
    )(pos_x, pos_y, pos_z, cen, nbr)
    return out.reshape(n_edges, N_OUT)


# speedup vs baseline: 4.7466x; 1.3457x over previous
import functools
import math

import jax
import jax.numpy as jnp
from jax import lax
from jax.experimental import pallas as pl
from jax.experimental.pallas import tpu as pltpu
from jax.experimental.pallas import tpu_sc as plsc

R_CUT = 5.0
N_MAX = 4
GAMMA = (N_MAX / R_CUT) ** 2
R_N = tuple(i * R_CUT / (N_MAX - 1) for i in range(N_MAX))
C0 = 0.28209479177387814
C1 = 0.4886025119029199
C2A = 1.0925484305920792
C2C = 0.31539156525252005
C2E = 0.5462742152960396
HALF_PI = math.pi / 2.0
U_SCALE = math.pi / (2.0 * R_CUT)
COS_COEFFS = (-1.0 / 87178291200.0, 1.0 / 479001600.0, -1.0 / 3628800.0,
              1.0 / 40320.0, -1.0 / 720.0, 1.0 / 24.0, -0.5, 1.0)

CHUNK = 128
LANES = 16
N_OUT = 36


def _expand_16(cbuf, nbuf, v):
    sl = pl.ds(v * LANES, LANES)
    x = nbuf[0][sl] - cbuf[0][sl]
    y = nbuf[1][sl] - cbuf[1][sl]
    z = nbuf[2][sl] - cbuf[2][sl]
    r2 = x * x + y * y + z * z + jnp.float32(1e-12)
    seed = (jnp.int32(0x5F3759DF)
            - (lax.bitcast_convert_type(r2, jnp.int32) >> 1))
    rinv = lax.bitcast_convert_type(seed, jnp.float32)
    for _ in range(3):
        rinv = rinv * (jnp.float32(1.5) - jnp.float32(0.5) * r2 * rinv * rinv)
    r = r2 * rinv
    rinv2 = rinv * rinv
    h = jnp.minimum(jnp.float32(U_SCALE) * r, jnp.float32(HALF_PI))
    h2 = h * h
    cc = jnp.full((LANES,), COS_COEFFS[0], jnp.float32)
    for co in COS_COEFFS[1:]:
        cc = cc * h2 + jnp.float32(co)
    fcut = jnp.where(r < jnp.float32(R_CUT), cc * cc, jnp.float32(0.0))
    rad = []
    for rn in R_N:
        dr = r - jnp.float32(rn)
        rad.append(jnp.exp(-jnp.float32(GAMMA) * dr * dr) * fcut)
    sh = [jnp.full((LANES,), C0, jnp.float32),
          jnp.float32(C1) * y * rinv,
          jnp.float32(C1) * z * rinv,
          jnp.float32(C1) * x * rinv,
          jnp.float32(C2A) * x * y * rinv2,
          jnp.float32(C2A) * y * z * rinv2,
          jnp.float32(C2C) * (jnp.float32(2.0) * z * z - x * x - y * y) * rinv2,
          jnp.float32(C2A) * x * z * rinv2,
          jnp.float32(C2E) * (x * x - y * y) * rinv2]
    return sh, rad


def _compute_chunk(cbuf, nbuf, out_stage):
    for v in range(CHUNK // LANES):
        sh, rad = _expand_16(cbuf, nbuf, v)
        ridx = lax.iota(jnp.int32, LANES) + jnp.int32(v * LANES)
        cidx = jnp.full((LANES,), 0, jnp.int32)
        one = jnp.full((LANES,), 1, jnp.int32)
        for s in sh:
            for rd in rad:
                plsc.store_scatter(out_stage, [ridx, cidx], s * rd)
                cidx = cidx + one


def _sc_body(pos_x, pos_y, pos_z, cen_hbm, nbr_hbm, out_hbm,
             ic0, ic1, in0, in1,
             c0x, c0y, c0z, n0x, n0y, n0z,
             c1x, c1y, c1z, n1x, n1y, n1z,
             os0, os1, shx, shy, shz,
             si0, si1, sg0, sg1, so0, so1, st0,
             *, n_rows, n_workers):
    nc = plsc.get_sparse_core_info().num_cores
    w = lax.axis_index("s") * nc + lax.axis_index("c")
    nk = (jnp.int32(n_rows - 1) - w) // jnp.int32(n_workers) + 1

    idx_c = (ic0, ic1)
    idx_n = (in0, in1)
    cbuf = ((c0x, c0y, c0z), (c1x, c1y, c1z))
    nbuf = ((n0x, n0y, n0z), (n1x, n1y, n1z))
    ostg = (os0, os1)
    sem_i = (si0, si1)
    sem_g = (sg0, sg1)
    sem_o = (so0, so1)
    tabs = (shx, shy, shz)

    @pl.when(lax.axis_index("s") == 0)
    def _():
        pltpu.async_copy(pos_x, shx, st0)
        pltpu.async_copy(pos_y, shy, st0)
        pltpu.async_copy(pos_z, shz, st0)
        pltpu.make_async_copy(pos_x, shx, st0).wait()
        pltpu.make_async_copy(pos_y, shy, st0).wait()
        pltpu.make_async_copy(pos_z, shz, st0).wait()

    plsc.subcore_barrier()

    def row_of(k):
        return w + k * jnp.int32(n_workers)

    def issue_idx(k, s):
        pltpu.async_copy(cen_hbm.at[row_of(k)], idx_c[s], sem_i[s])
        pltpu.async_copy(nbr_hbm.at[row_of(k)], idx_n[s], sem_i[s])

    def wait_idx(s):
        pltpu.make_async_copy(cen_hbm.at[0], idx_c[s], sem_i[s]).wait()
        pltpu.make_async_copy(nbr_hbm.at[0], idx_n[s], sem_i[s]).wait()

    def issue_gathers(s):
        for tab, dst in zip(tabs, cbuf[s]):
            pltpu.async_copy(tab.at[idx_c[s]], dst, sem_g[s])
        for tab, dst in zip(tabs, nbuf[s]):
            pltpu.async_copy(tab.at[idx_n[s]], dst, sem_g[s])

    def wait_gathers(s):
        for tab, dst in zip(tabs, cbuf[s]):
            pltpu.make_async_copy(tab.at[idx_c[s]], dst, sem_g[s]).wait()
        for tab, dst in zip(tabs, nbuf[s]):
            pltpu.make_async_copy(tab.at[idx_n[s]], dst, sem_g[s]).wait()

    def issue_out(k, s):
        pltpu.async_copy(
            ostg[s], out_hbm.at[pl.ds(row_of(k) * CHUNK, CHUNK)], sem_o[s])

    def wait_out(s):
        pltpu.make_async_copy(
            ostg[s], out_hbm.at[pl.ds(0, CHUNK)], sem_o[s]).wait()

    issue_idx(0, 0)

    @pl.when(nk > 1)
    def _():
        issue_idx(1, 1)

    wait_idx(0)
    issue_gathers(0)

    @pl.loop(0, nk, step=2)
    def _pair(k0):
        for s in (0, 1):
            k = k0 + jnp.int32(s)

            @pl.when(k < nk)
            def _():
                wait_gathers(s)

                @pl.when(k + 2 < nk)
                def _():
                    issue_idx(k + 2, s)

                @pl.when(k + 1 < nk)
                def _():
                    wait_idx(1 - s)
                    issue_gathers(1 - s)

                @pl.when(k >= 2)
                def _():
                    wait_out(s)

                _compute_chunk(cbuf[s], nbuf[s], ostg[s])
                issue_out(k, s)

    @pl.when(nk >= 2)
    def _():
        wait_out(0)
        wait_out(1)

    @pl.when(nk == 1)
    def _():
        wait_out(0)


def kernel(positions, edge_index):
    n_edges = edge_index.shape[1]
    assert n_edges % CHUNK == 0
    n_rows = n_edges // CHUNK

    pos_x = positions[:, 0]
    pos_y = positions[:, 1]
    pos_z = positions[:, 2]
    cen = edge_index[0].reshape(n_rows, CHUNK)
    nbr = edge_index[1].reshape(n_rows, CHUNK)

    info = plsc.get_sparse_core_info()
    n_workers = info.num_cores * info.num_subcores
    mesh = plsc.VectorSubcoreMesh(core_axis_name="c", subcore_axis_name="s")
    body = functools.partial(_sc_body, n_rows=n_rows, n_workers=n_workers)
    idx_t = pltpu.VMEM((CHUNK,), jnp.int32)
    buf_t = pltpu.VMEM((CHUNK,), jnp.float32)
    out_t = pltpu.VMEM((CHUNK, N_OUT), jnp.float32)
    out = pl.kernel(
        body,
        out_type=jax.ShapeDtypeStruct((n_edges, N_OUT), jnp.float32),
        mesh=mesh,
        scratch_types=(
            [idx_t] * 4 + [buf_t] * 12 + [out_t] * 2
            + [pltpu.VMEM_SHARED((positions.shape[0],), jnp.float32)] * 3
            + [pltpu.SemaphoreType.DMA] * 7
        ),
        compiler_params=pltpu.CompilerParams(needs_layout_passes=False),
    )(pos_x, pos_y, pos_z, cen, nbr)
    return out
